# baseline (speedup 1.0000x reference)
import jax
import jax.numpy as jnp
from jax import lax
from jax.experimental import pallas as pl
from jax.experimental.pallas import tpu as pltpu
from jax.experimental.pallas import tpu_sc as plsc

VOCAB = 1000000
D = 32
BATCH = 16384
FIELDS = 26
B = BATCH * FIELDS

NC = 2
NS = 16
NW = NC * NS
PER_W = B // NW
G = 128
K = 13
SC_ROWS = G * K
NSC = PER_W // SC_ROWS
NCH = PER_W // G


def _repack_body(tableT_hbm, tail_hbm, packed_hbm, a_v, b_v):
    wid = lax.axis_index("s") * NC + lax.axis_index("c")
    row_lo = lax.iota(jnp.int32, 16)
    row_hi = row_lo + 16
    trips = 244 + jnp.where(wid < 4, 1, 0)

    def transpose_cols(ncols):
        def col(v, c2):
            cols = jnp.full((16,), v, jnp.int32)
            lo = plsc.load_gather(a_v, [row_lo, cols])
            hi = plsc.load_gather(a_v, [row_hi, cols])
            b_v[pl.ds(v * 32, 16)] = lo
            b_v[pl.ds(v * 32 + 16, 16)] = hi
            return c2
        lax.fori_loop(0, ncols, col, 0)

    def blk_body(i, carry):
        blk = i * 32 + wid
        c0 = blk * 128
        pltpu.sync_copy(tableT_hbm.at[:, pl.ds(c0, 128)], a_v)
        transpose_cols(128)
        pltpu.sync_copy(b_v, packed_hbm.at[pl.ds(c0 * 32, 4096)])
        return carry
    lax.fori_loop(0, trips, blk_body, 0)

    @pl.when(wid == 4)
    def _tail():
        c0 = (VOCAB // 128) * 128
        pltpu.sync_copy(tail_hbm, b_v.at[pl.ds(0, 2048)])
        pltpu.sync_copy(b_v.at[pl.ds(0, 2048)],
                        packed_hbm.at[pl.ds(c0 * 32, 2048)])


def _sc_body(table_hbm, idx_hbm, val_hbm, out_hbm, idx_v, val_v, rows_v, sem):
    wid = lax.axis_index("s") * NC + lax.axis_index("c")
    base = wid * PER_W
    pltpu.sync_copy(idx_hbm.at[wid], idx_v)
    pltpu.sync_copy(val_hbm.at[wid], val_v)

    def superchunk(s, carry):
        copies = []
        for j in range(K):
            copies.append(pltpu.async_copy(
                table_hbm.at[idx_v.at[s * K + j]],
                rows_v.at[pl.ds(j * G, G)],
                sem))
        for c in copies:
            c.wait()

        def group(g, carry2):
            v16 = val_v[pl.ds(s * SC_ROWS + g * 16, 16)]
            for l in range(16):
                i = g * 16 + l
                v = v16[l]
                r0 = rows_v[i, pl.ds(0, 16)]
                rows_v[i, pl.ds(0, 16)] = r0 * v
                r1 = rows_v[i, pl.ds(16, 16)]
                rows_v[i, pl.ds(16, 16)] = r1 * v
            return carry2
        lax.fori_loop(0, SC_ROWS // 16, group, 0)

        pltpu.sync_copy(rows_v,
                        out_hbm.at[pl.ds(base + s * SC_ROWS, SC_ROWS)])
        return carry
    lax.fori_loop(0, NSC, superchunk, 0)


@jax.jit
def kernel(embedding, inputs_id, inputs_value):
    ids = inputs_id.astype(jnp.int32).reshape(NW, NCH, G)
    vals = inputs_value.reshape(NW, PER_W)
    mesh = plsc.VectorSubcoreMesh(core_axis_name="c", subcore_axis_name="s")
    packed = pl.kernel(
        _repack_body,
        mesh=mesh,
        compiler_params=pltpu.CompilerParams(needs_layout_passes=False),
        out_type=jax.ShapeDtypeStruct((VOCAB * D,), jnp.float32),
        scratch_types=[
            pltpu.VMEM((32, 128), jnp.float32),
            pltpu.VMEM((4096,), jnp.float32),
        ],
    )(embedding.T, embedding[(VOCAB // 128) * 128:, :].reshape(-1))
    table_lin = packed.reshape(VOCAB, D)
    out = pl.kernel(
        _sc_body,
        mesh=mesh,
        compiler_params=pltpu.CompilerParams(use_tc_tiling_on_sc=False),
        out_type=jax.ShapeDtypeStruct((B, D), jnp.float32),
        scratch_types=[
            pltpu.VMEM((NCH, G), jnp.int32),
            pltpu.VMEM((PER_W,), jnp.float32),
            pltpu.VMEM((SC_ROWS, D), jnp.float32),
            pltpu.SemaphoreType.DMA,
        ],
    )(table_lin, ids, vals)
    return out.reshape(BATCH, FIELDS, D)


# speedup vs baseline: 1.7512x; 1.7512x over previous
import jax
import jax.numpy as jnp
from jax import lax
from jax.experimental import pallas as pl
from jax.experimental.pallas import tpu as pltpu
from jax.experimental.pallas import tpu_sc as plsc

VOCAB = 1000000
D = 32
BATCH = 16384
FIELDS = 26
B = BATCH * FIELDS

NC = 2
NS = 16
NW = NC * NS
PER_W = B // NW
G = 128
K = 13
SC_ROWS = G * K
NSC = PER_W // SC_ROWS
NCH = PER_W // G


def _repack_body(tableT_hbm, tail_hbm, packed_hbm, a_v, b_v, sem):
    wid = lax.axis_index("s") * NC + lax.axis_index("c")
    lane = lax.iota(jnp.int32, 16)
    lane32 = lane * 32
    rots = [(lane + j) % 16 for j in range(16)]

    def transpose_group(nq):
        def qbody(q, c2):
            k = jnp.full((16,), q // 8, jnp.int32)
            col = (q % 8) * 16 + lane
            for h in range(2):
                base = lane32 + (q * 512 + h * 16)
                for j in range(16):
                    row = rots[j] + (h * 16)
                    vec = plsc.load_gather(a_v, [k, row, col])
                    plsc.store_scatter(b_v, [base + rots[j]], vec)
            return c2
        lax.fori_loop(0, nq, qbody, 0)

    def grp_body(i, carry):
        grp = i * 32 + wid
        c0 = grp * 512
        copies = [pltpu.async_copy(
            tableT_hbm.at[:, pl.ds(c0 + kk * 128, 128)],
            a_v.at[kk], sem) for kk in range(4)]
        for c in copies:
            c.wait()
        transpose_group(32)
        pltpu.sync_copy(b_v, packed_hbm.at[pl.ds(c0 * 32, 16384)])
        return carry
    lax.fori_loop(0, 61, grp_body, 0)

    @pl.when(wid < 4)
    def _extra():
        c0 = (7808 + wid) * 128
        pltpu.sync_copy(tableT_hbm.at[:, pl.ds(c0, 128)], a_v.at[0])
        transpose_group(8)
        pltpu.sync_copy(b_v.at[pl.ds(0, 4096)],
                        packed_hbm.at[pl.ds(c0 * 32, 4096)])

    @pl.when(wid == 4)
    def _tail():
        c0 = (VOCAB // 128) * 128
        pltpu.sync_copy(tail_hbm, b_v.at[pl.ds(0, 2048)])
        pltpu.sync_copy(b_v.at[pl.ds(0, 2048)],
                        packed_hbm.at[pl.ds(c0 * 32, 2048)])


def _sc_body(table_hbm, idx_hbm, val_hbm, out_hbm, idx_v, val_v, rows_v, sem):
    wid = lax.axis_index("s") * NC + lax.axis_index("c")
    base = wid * PER_W
    pltpu.sync_copy(idx_hbm.at[wid], idx_v)
    pltpu.sync_copy(val_hbm.at[wid], val_v)

    def superchunk(s, carry):
        copies = []
        for j in range(K):
            copies.append(pltpu.async_copy(
                table_hbm.at[idx_v.at[s * K + j]],
                rows_v.at[pl.ds(j * G, G)],
                sem))
        for c in copies:
            c.wait()

        def group(g, carry2):
            v16 = val_v[pl.ds(s * SC_ROWS + g * 16, 16)]
            for l in range(16):
                i = g * 16 + l
                v = v16[l]
                r0 = rows_v[i, pl.ds(0, 16)]
                rows_v[i, pl.ds(0, 16)] = r0 * v
                r1 = rows_v[i, pl.ds(16, 16)]
                rows_v[i, pl.ds(16, 16)] = r1 * v
            return carry2
        lax.fori_loop(0, SC_ROWS // 16, group, 0)

        pltpu.sync_copy(rows_v,
                        out_hbm.at[pl.ds(base + s * SC_ROWS, SC_ROWS)])
        return carry
    lax.fori_loop(0, NSC, superchunk, 0)


@jax.jit
def kernel(embedding, inputs_id, inputs_value):
    ids = inputs_id.astype(jnp.int32).reshape(NW, NCH, G)
    vals = inputs_value.reshape(NW, PER_W)
    mesh = plsc.VectorSubcoreMesh(core_axis_name="c", subcore_axis_name="s")
    packed = pl.kernel(
        _repack_body,
        mesh=mesh,
        compiler_params=pltpu.CompilerParams(needs_layout_passes=False),
        out_type=jax.ShapeDtypeStruct((VOCAB * D,), jnp.float32),
        scratch_types=[
            pltpu.VMEM((4, 32, 128), jnp.float32),
            pltpu.VMEM((16384,), jnp.float32),
            pltpu.SemaphoreType.DMA,
        ],
    )(embedding.T, embedding[(VOCAB // 128) * 128:, :].reshape(-1))
    table_lin = packed.reshape(VOCAB, D)
    out = pl.kernel(
        _sc_body,
        mesh=mesh,
        compiler_params=pltpu.CompilerParams(use_tc_tiling_on_sc=False),
        out_type=jax.ShapeDtypeStruct((B, D), jnp.float32),
        scratch_types=[
            pltpu.VMEM((NCH, G), jnp.int32),
            pltpu.VMEM((PER_W,), jnp.float32),
            pltpu.VMEM((SC_ROWS, D), jnp.float32),
            pltpu.SemaphoreType.DMA,
        ],
    )(table_lin, ids, vals)
    return out.reshape(BATCH, FIELDS, D)


# speedup vs baseline: 2.1483x; 1.2267x over previous
import jax
import jax.numpy as jnp
from jax import lax
from jax.experimental import pallas as pl
from jax.experimental.pallas import tpu as pltpu
from jax.experimental.pallas import tpu_sc as plsc

VOCAB = 1000000
D = 32
BATCH = 16384
FIELDS = 26
B = BATCH * FIELDS

NC = 2
NS = 16
NW = NC * NS
PER_W = B // NW
G = 128
K = 13
SC_ROWS = G * K
NSC = PER_W // SC_ROWS
NCH = PER_W // G


def _repack_body(tableT_hbm, tail_hbm, packed_hbm, a_v, b_v, sem_in, sem_out):
    wid = lax.axis_index("s") * NC + lax.axis_index("c")
    lane = lax.iota(jnp.int32, 16)
    lane32 = lane * 32
    rots = [(lane + j) % 16 for j in range(16)]

    def transpose_group(p, nq):
        pvec = jnp.full((16,), p, jnp.int32)
        pbase = lane32 + p * 16384

        def qbody(q, c2):
            k = jnp.full((16,), q // 8, jnp.int32)
            col = (q % 8) * 16 + lane
            for h in range(2):
                base = pbase + (q * 512 + h * 16)
                for j in range(16):
                    row = rots[j] + (h * 16)
                    vec = plsc.load_gather(a_v, [pvec, k, row, col])
                    plsc.store_scatter(b_v, [base + rots[j]], vec)
            return c2
        lax.fori_loop(0, nq, qbody, 0)

    def in_copies(i, p):
        grp = i * 32 + wid
        c0 = grp * 512
        return [pltpu.make_async_copy(
            tableT_hbm.at[:, pl.ds(c0 + kk * 128, 128)],
            a_v.at[p, kk], sem_in.at[p]) for kk in range(4)]

    def out_copy(i, p):
        grp = i * 32 + wid
        c0 = grp * 512
        return pltpu.make_async_copy(
            b_v.at[pl.ds(p * 16384, 16384)],
            packed_hbm.at[pl.ds(c0 * 32, 16384)], sem_out.at[p])

    for c in in_copies(0, 0):
        c.start()

    def grp_body(i, carry):
        p = lax.rem(i, 2)
        for c in in_copies(i, p):
            c.wait()

        @pl.when(i + 1 < 61)
        def _prefetch():
            for c in in_copies(i + 1, 1 - p):
                c.start()

        @pl.when(i >= 2)
        def _drain():
            out_copy(i - 2, p).wait()
        transpose_group(p, 32)
        out_copy(i, p).start()
        return carry
    lax.fori_loop(0, 61, grp_body, 0)
    out_copy(59, 1).wait()
    out_copy(60, 0).wait()

    @pl.when(wid < 4)
    def _extra():
        c0 = (7808 + wid) * 128
        pltpu.sync_copy(tableT_hbm.at[:, pl.ds(c0, 128)], a_v.at[0, 0])
        transpose_group(0, 8)
        pltpu.sync_copy(b_v.at[pl.ds(0, 4096)],
                        packed_hbm.at[pl.ds(c0 * 32, 4096)])

    @pl.when(wid == 4)
    def _tail():
        c0 = (VOCAB // 128) * 128
        pltpu.sync_copy(tail_hbm, b_v.at[pl.ds(0, 2048)])
        pltpu.sync_copy(b_v.at[pl.ds(0, 2048)],
                        packed_hbm.at[pl.ds(c0 * 32, 2048)])


def _sc_body(table_hbm, idx_hbm, val_hbm, out_hbm, idx_v, val_v, rows_v, sem):
    wid = lax.axis_index("s") * NC + lax.axis_index("c")
    base = wid * PER_W
    pltpu.sync_copy(idx_hbm.at[wid], idx_v)
    pltpu.sync_copy(val_hbm.at[wid], val_v)

    def superchunk(s, carry):
        copies = []
        for j in range(K):
            copies.append(pltpu.async_copy(
                table_hbm.at[idx_v.at[s * K + j]],
                rows_v.at[pl.ds(j * G, G)],
                sem))
        for c in copies:
            c.wait()

        def group(g, carry2):
            v16 = val_v[pl.ds(s * SC_ROWS + g * 16, 16)]
            for l in range(16):
                i = g * 16 + l
                v = v16[l]
                r0 = rows_v[i, pl.ds(0, 16)]
                rows_v[i, pl.ds(0, 16)] = r0 * v
                r1 = rows_v[i, pl.ds(16, 16)]
                rows_v[i, pl.ds(16, 16)] = r1 * v
            return carry2
        lax.fori_loop(0, SC_ROWS // 16, group, 0)

        pltpu.sync_copy(rows_v,
                        out_hbm.at[pl.ds(base + s * SC_ROWS, SC_ROWS)])
        return carry
    lax.fori_loop(0, NSC, superchunk, 0)


@jax.jit
def kernel(embedding, inputs_id, inputs_value):
    ids = inputs_id.astype(jnp.int32).reshape(NW, NCH, G)
    vals = inputs_value.reshape(NW, PER_W)
    mesh = plsc.VectorSubcoreMesh(core_axis_name="c", subcore_axis_name="s")
    packed = pl.kernel(
        _repack_body,
        mesh=mesh,
        compiler_params=pltpu.CompilerParams(needs_layout_passes=False),
        out_type=jax.ShapeDtypeStruct((VOCAB * D,), jnp.float32),
        scratch_types=[
            pltpu.VMEM((2, 4, 32, 128), jnp.float32),
            pltpu.VMEM((32768,), jnp.float32),
            pltpu.SemaphoreType.DMA((2,)),
            pltpu.SemaphoreType.DMA((2,)),
        ],
    )(embedding.T, embedding[(VOCAB // 128) * 128:, :].reshape(-1))
    table_lin = packed.reshape(VOCAB, D)
    out = pl.kernel(
        _sc_body,
        mesh=mesh,
        compiler_params=pltpu.CompilerParams(use_tc_tiling_on_sc=False),
        out_type=jax.ShapeDtypeStruct((B, D), jnp.float32),
        scratch_types=[
            pltpu.VMEM((NCH, G), jnp.int32),
            pltpu.VMEM((PER_W,), jnp.float32),
            pltpu.VMEM((SC_ROWS, D), jnp.float32),
            pltpu.SemaphoreType.DMA,
        ],
    )(table_lin, ids, vals)
    return out.reshape(BATCH, FIELDS, D)


# speedup vs baseline: 3.0410x; 1.4156x over previous
import jax
import jax.numpy as jnp
from jax import lax
from jax.experimental import pallas as pl
from jax.experimental.pallas import tpu as pltpu
from jax.experimental.pallas import tpu_sc as plsc

VOCAB = 1000000
D = 32
BATCH = 16384
FIELDS = 26
B = BATCH * FIELDS

NC = 2
NS = 16
NW = NC * NS
PER_W = B // NW
G = 128
K = 13
SC_ROWS = G * K
NSC = PER_W // SC_ROWS
NCH = PER_W // G


def _repack_body(tableT_hbm, tail_hbm, packed_hbm, a_v, b_v, sem_in, sem_out):
    wid = lax.axis_index("s") * NC + lax.axis_index("c")
    lane = lax.iota(jnp.int32, 16)
    lane32 = lane * 32
    rots = [(lane + j) % 16 for j in range(16)]

    def transpose_group(p, nq):
        pvec = jnp.full((16,), p, jnp.int32)
        pbase = lane32 + p * 16384

        def qbody(q, c2):
            k = jnp.full((16,), q // 8, jnp.int32)
            col = (q % 8) * 16 + lane
            for h in range(2):
                base = pbase + (q * 512 + h * 16)
                for j in range(16):
                    row = rots[j] + (h * 16)
                    vec = plsc.load_gather(a_v, [pvec, k, row, col])
                    plsc.store_scatter(b_v, [base + rots[j]], vec)
            return c2
        lax.fori_loop(0, nq, qbody, 0)

    def in_copies(i, p):
        grp = i * 32 + wid
        c0 = grp * 512
        return [pltpu.make_async_copy(
            tableT_hbm.at[:, pl.ds(c0 + kk * 128, 128)],
            a_v.at[p, kk], sem_in.at[p]) for kk in range(4)]

    def out_copy(i, p):
        grp = i * 32 + wid
        c0 = grp * 512
        return pltpu.make_async_copy(
            b_v.at[pl.ds(p * 16384, 16384)],
            packed_hbm.at[pl.ds(c0 * 32, 16384)], sem_out.at[p])

    for c in in_copies(0, 0):
        c.start()

    def grp_body(i, carry):
        p = lax.rem(i, 2)
        for c in in_copies(i, p):
            c.wait()

        @pl.when(i + 1 < 61)
        def _prefetch():
            for c in in_copies(i + 1, 1 - p):
                c.start()

        @pl.when(i >= 2)
        def _drain():
            out_copy(i - 2, p).wait()
        transpose_group(p, 32)
        out_copy(i, p).start()
        return carry
    lax.fori_loop(0, 61, grp_body, 0)
    out_copy(59, 1).wait()
    out_copy(60, 0).wait()

    @pl.when(wid < 4)
    def _extra():
        c0 = (7808 + wid) * 128
        pltpu.sync_copy(tableT_hbm.at[:, pl.ds(c0, 128)], a_v.at[0, 0])
        transpose_group(0, 8)
        pltpu.sync_copy(b_v.at[pl.ds(0, 4096)],
                        packed_hbm.at[pl.ds(c0 * 32, 4096)])

    @pl.when(wid == 4)
    def _tail():
        c0 = (VOCAB // 128) * 128
        pltpu.sync_copy(tail_hbm, b_v.at[pl.ds(0, 2048)])
        pltpu.sync_copy(b_v.at[pl.ds(0, 2048)],
                        packed_hbm.at[pl.ds(c0 * 32, 2048)])


def _gather_body(table_hbm, idsT_hbm, valsT_hbm, out_hbm,
                 ids_v, vals_v, rows_v, stage_v, sem_g, sem_o):
    wid = lax.axis_index("s") * NC + lax.axis_index("c")
    b0 = wid * 512
    lane = lax.iota(jnp.int32, 16)
    lane32 = lane * 32
    rots = [(lane + j) % 16 for j in range(16)]
    dparts = [(r >> 3) * 4096 + (r & 7) * 128 for r in rots]

    pltpu.sync_copy(idsT_hbm.at[:, pl.ds(b0, 512)], ids_v)
    pltpu.sync_copy(valsT_hbm.at[:, pl.ds(b0, 512)], vals_v)

    def g_copies(f, p):
        return [pltpu.make_async_copy(
            table_hbm.at[ids_v.at[f, pl.ds(c * 128, 128)]],
            rows_v.at[p, pl.ds(c * 128, 128)], sem_g.at[p])
            for c in range(4)]

    def out_copies(f, p):
        cps = []
        for dt in range(4):
            off = f * 524288 + dt * 131072 + wid * 4096
            cps.append(pltpu.make_async_copy(
                stage_v.at[pl.ds(p * 16384 + dt * 4096, 4096)],
                out_hbm.at[pl.ds(off, 4096)], sem_o.at[p]))
        return cps

    def transpose_scale(f, p):
        pvec = jnp.full((16,), p, jnp.int32)

        def qbody(q, c2):
            vv = vals_v[f, pl.ds(q * 16, 16)]
            bvec = q * 16 + lane
            sbase = (p * 16384 + (q >> 3) * 1024 + (q & 7) * 16) + lane
            for h in range(2):
                sb2 = sbase + h * 8192
                for j in range(16):
                    dvec = rots[j] + h * 16
                    vec = plsc.load_gather(rows_v, [pvec, bvec, dvec])
                    plsc.store_scatter(stage_v, [sb2 + dparts[j]], vec * vv)
            return c2
        lax.fori_loop(0, 32, qbody, 0)

    for cp in g_copies(0, 0):
        cp.start()

    def fbody(f, carry):
        p = lax.rem(f, 2)
        for cp in g_copies(f, p):
            cp.wait()

        @pl.when(f + 1 < FIELDS)
        def _prefetch():
            for cp in g_copies(f + 1, 1 - p):
                cp.start()

        @pl.when(f >= 2)
        def _drain():
            for cp in out_copies(f - 2, p):
                cp.wait()
        transpose_scale(f, p)
        for cp in out_copies(f, p):
            cp.start()
        return carry
    lax.fori_loop(0, FIELDS, fbody, 0)
    for cp in out_copies(FIELDS - 2, 0):
        cp.wait()
    for cp in out_copies(FIELDS - 1, 1):
        cp.wait()


@jax.jit
def kernel(embedding, inputs_id, inputs_value):
    idsT = inputs_id.astype(jnp.int32).T
    valsT = inputs_value.T
    mesh = plsc.VectorSubcoreMesh(core_axis_name="c", subcore_axis_name="s")
    packed = pl.kernel(
        _repack_body,
        mesh=mesh,
        compiler_params=pltpu.CompilerParams(needs_layout_passes=False),
        out_type=jax.ShapeDtypeStruct((VOCAB * D,), jnp.float32),
        scratch_types=[
            pltpu.VMEM((2, 4, 32, 128), jnp.float32),
            pltpu.VMEM((32768,), jnp.float32),
            pltpu.SemaphoreType.DMA((2,)),
            pltpu.SemaphoreType.DMA((2,)),
        ],
    )(embedding.T, embedding[(VOCAB // 128) * 128:, :].reshape(-1))
    table_lin = packed.reshape(VOCAB, D)
    out_flat = pl.kernel(
        _gather_body,
        mesh=mesh,
        compiler_params=pltpu.CompilerParams(
            use_tc_tiling_on_sc=False, needs_layout_passes=False),
        out_type=jax.ShapeDtypeStruct((B * D,), jnp.float32),
        scratch_types=[
            pltpu.VMEM((FIELDS, 512), jnp.int32),
            pltpu.VMEM((FIELDS, 512), jnp.float32),
            pltpu.VMEM((2, 512, D), jnp.float32),
            pltpu.VMEM((32768,), jnp.float32),
            pltpu.SemaphoreType.DMA((2,)),
            pltpu.SemaphoreType.DMA((2,)),
        ],
    )(table_lin, idsT, valsT)
    out5 = out_flat.reshape(FIELDS, 4, 128, 8, 128)
    return out5.transpose(2, 4, 0, 1, 3).reshape(BATCH, FIELDS, D)
